# Initial kernel scaffold; baseline (speedup 1.0000x reference)
import jax
import jax.numpy as jnp
from jax.experimental import pallas as pl


def kernel(x, reg):
    raise NotImplementedError("write your pallas kernel here")


# speedup vs baseline: 1.6800x; 1.6800x over previous
import functools

import jax
import jax.numpy as jnp
from jax import lax
from jax.experimental import pallas as pl
from jax.experimental.pallas import tpu as pltpu
from jax.experimental.pallas import tpu_sc as plsc


def _build(B, C, H, W, Hr, Wr):
    ph = (Hr - H) // 2
    pw = (Wr - W) // 2
    C2 = 2 * C

    NC, NS = 2, 16
    NW = NC * NS
    n_x_planes = B * C
    PX = n_x_planes // NW
    PZ = n_x_planes // NW
    Hh = Hr // 2
    WL = Wr // 16

    mesh = plsc.VectorSubcoreMesh(core_axis_name="c", subcore_axis_name="s")

    @functools.partial(
        pl.kernel,
        mesh=mesh,
        out_type=jax.ShapeDtypeStruct((B, C2, Hr, Wr), jnp.float32),
        scratch_types=[
            pltpu.VMEM((Hr, Wr), jnp.float32),
            pltpu.VMEM((Hr, Wr), jnp.float32),
            pltpu.VMEM((Hh, Wr), jnp.float32),
            pltpu.SemaphoreType.DMA,
            pltpu.SemaphoreType.DMA,
            pltpu.SemaphoreType.DMA,
            pltpu.SemaphoreType.DMA,
            pltpu.SemaphoreType.DMA,
        ],
        compiler_params=pltpu.CompilerParams(use_tc_tiling_on_sc=False),
    )
    def run(x_hbm, out_hbm, pbuf0, pbuf1, zbuf, si0, si1, so0, so1, sz):
        wid = lax.axis_index("s") * NC + lax.axis_index("c")
        zero16 = jnp.zeros((16,), jnp.float32)

        def zrow(r, _):
            for j in range(WL):
                zbuf[r, pl.ds(j * 16, 16)] = zero16
            return 0
        lax.fori_loop(0, Hh, zrow, 0)

        for buf in (pbuf0, pbuf1):
            def trow(r, _, buf=buf):
                rr = jnp.where(r < ph, r, r + (Hr - 2 * ph))
                for j in range(WL):
                    buf[rr, pl.ds(j * 16, 16)] = zero16
                return 0
            lax.fori_loop(0, 2 * ph, trow, 0)

            def srow(r, _, buf=buf):
                buf[ph + r, pl.ds(0, pw)] = zero16[:pw] if pw != 16 else zero16
                buf[ph + r, pl.ds(Wr - pw, pw)] = (
                    zero16[:pw] if pw != 16 else zero16)
                return 0
            lax.fori_loop(0, H, srow, 0)

        bufs = (pbuf0, pbuf1)
        sin = (si0, si1)
        sout = (so0, so1)

        def bc_x(i):
            p = wid * PX + i
            return p // C, p % C

        def bc_z(i):
            p = wid * PZ + i
            return p // C, C + p % C

        def copy_in(i, k):
            b, c = bc_x(i)
            return pltpu.async_copy(
                x_hbm.at[b, c],
                bufs[k].at[pl.ds(ph, H), pl.ds(pw, W)],
                sin[k])

        zhandles = []
        in_h = [None, None]
        out_h = [None, None]
        in_h[0] = copy_in(0, 0)
        for i in range(PX):
            k = i % 2
            kn = (i + 1) % 2
            b, c = bc_z(i)
            zhandles.append(pltpu.async_copy(
                zbuf, out_hbm.at[b, c, pl.ds(0, Hh)], sz))
            zhandles.append(pltpu.async_copy(
                zbuf, out_hbm.at[b, c, pl.ds(Hh, Hh)], sz))
            if i + 1 < PX:
                if out_h[kn] is not None:
                    out_h[kn].wait()
                in_h[kn] = copy_in(i + 1, kn)
            in_h[k].wait()
            b, c = bc_x(i)
            out_h[k] = pltpu.async_copy(bufs[k], out_hbm.at[b, c], sout[k])
        for k in range(2):
            if out_h[k] is not None:
                out_h[k].wait()
        for h in zhandles:
            h.wait()

    return run


def kernel(x, reg):
    B, C, H, W = x.shape
    Hr, Wr = reg.shape[2], reg.shape[3]
    return _build(B, C, H, W, Hr, Wr)(x)


# speedup vs baseline: 1.9786x; 1.1777x over previous
import functools

import jax
import jax.numpy as jnp
from jax import lax
from jax.experimental import pallas as pl
from jax.experimental.pallas import tpu as pltpu
from jax.experimental.pallas import tpu_sc as plsc


def _build(B, C, H, W, Hr, Wr):
    ph = (Hr - H) // 2
    pw = (Wr - W) // 2
    C2 = 2 * C

    NC, NS = 2, 16
    NW = NC * NS
    PZ = (B * C) // NW
    Hh = Hr // 2

    mesh = plsc.VectorSubcoreMesh(core_axis_name="c", subcore_axis_name="s")

    @functools.partial(
        pl.kernel,
        mesh=mesh,
        out_type=jax.ShapeDtypeStruct((B, C2, Hr, Wr), jnp.float32),
        scratch_types=[
            pltpu.VMEM((Hh, Wr), jnp.float32),
            pltpu.SemaphoreType.DMA,
        ],
        compiler_params=pltpu.CompilerParams(use_tc_tiling_on_sc=True),
    )
    def fill_zero_planes(zp_hbm, out_hbm, zbuf, sem):
        wid = lax.axis_index("s") * NC + lax.axis_index("c")
        pltpu.sync_copy(zp_hbm, zbuf)
        handles = []
        for i in range(PZ):
            p = wid * PZ + i
            b = p // C
            c = C + p % C
            handles.append(pltpu.async_copy(
                zbuf, out_hbm.at[b, c, pl.ds(0, Hh)], sem))
            handles.append(pltpu.async_copy(
                zbuf, out_hbm.at[b, c, pl.ds(Hh, Hh)], sem))
        for h in handles:
            h.wait()

    def pad_body(x_ref, buf_ref, out_ref):
        del buf_ref
        out_ref[0, 0] = jnp.zeros((Hr, Wr), jnp.float32)
        out_ref[0, 0, ph:ph + H, pw:pw + W] = x_ref[0, 0]

    pad_x = pl.pallas_call(
        pad_body,
        grid=(B, C),
        in_specs=[
            pl.BlockSpec((1, 1, H, W), lambda b, c: (b, c, 0, 0)),
            pl.BlockSpec(memory_space=pl.ANY),
        ],
        out_specs=pl.BlockSpec((1, 1, Hr, Wr), lambda b, c: (b, c, 0, 0)),
        out_shape=jax.ShapeDtypeStruct((B, C2, Hr, Wr), jnp.float32),
        input_output_aliases={1: 0},
    )

    def run(x):
        zp = jnp.zeros((Hh, Wr), jnp.float32)
        buf = fill_zero_planes(zp)
        return pad_x(x, buf)

    return run


def kernel(x, reg):
    B, C, H, W = x.shape
    Hr, Wr = reg.shape[2], reg.shape[3]
    return _build(B, C, H, W, Hr, Wr)(x)


# speedup vs baseline: 4.7407x; 2.3960x over previous
import functools

import jax
import jax.numpy as jnp
from jax import lax
from jax.experimental import pallas as pl
from jax.experimental.pallas import tpu as pltpu
from jax.experimental.pallas import tpu_sc as plsc


def _build(B, C, H, W, Hr, Wr):
    ph = (Hr - H) // 2
    pw = (Wr - W) // 2
    C2 = 2 * C

    NC, NS = 2, 16
    NW = NC * NS
    PZ = (B * C) // NW
    Hh = Hr // 2

    mesh = plsc.VectorSubcoreMesh(core_axis_name="c", subcore_axis_name="s")

    @functools.partial(
        pl.kernel,
        mesh=mesh,
        out_type=jax.ShapeDtypeStruct((B, C2, Hr, Wr), jnp.float32),
        scratch_types=[
            pltpu.VMEM((Hh, Wr), jnp.float32),
            pltpu.SemaphoreType.DMA,
        ],
        compiler_params=pltpu.CompilerParams(use_tc_tiling_on_sc=True),
    )
    def fill_zero_planes(zp_hbm, out_hbm, zbuf, sem):
        wid = lax.axis_index("s") * NC + lax.axis_index("c")
        pltpu.sync_copy(zp_hbm, zbuf)
        handles = []
        for i in range(PZ):
            p = wid * PZ + i
            b = p // C
            c = C + p % C
            handles.append(pltpu.async_copy(
                zbuf, out_hbm.at[b, c, pl.ds(0, Hh)], sem))
            handles.append(pltpu.async_copy(
                zbuf, out_hbm.at[b, c, pl.ds(Hh, Hh)], sem))
        for h in handles:
            h.wait()

    CB = 8

    def pad_body(x_ref, buf_ref, out_ref):
        del buf_ref
        for ch in range(CB):
            out_ref[0, ch] = jnp.zeros((Hr, Wr), jnp.float32)
            out_ref[0, ch, ph:ph + H, pw:pw + W] = x_ref[0, ch]

    pad_x = pl.pallas_call(
        pad_body,
        grid=(B, C // CB),
        in_specs=[
            pl.BlockSpec((1, CB, H, W), lambda b, g: (b, g, 0, 0)),
            pl.BlockSpec(memory_space=pl.ANY),
        ],
        out_specs=pl.BlockSpec((1, CB, Hr, Wr), lambda b, g: (b, g, 0, 0)),
        out_shape=jax.ShapeDtypeStruct((B, C2, Hr, Wr), jnp.float32),
        input_output_aliases={1: 0},
    )

    def run(x):
        zp = jnp.zeros((Hh, Wr), jnp.float32)
        buf = fill_zero_planes(zp)
        return pad_x(x, buf)

    return run


def kernel(x, reg):
    B, C, H, W = x.shape
    Hr, Wr = reg.shape[2], reg.shape[3]
    return _build(B, C, H, W, Hr, Wr)(x)


# speedup vs baseline: 5.1102x; 1.0779x over previous
import functools

import jax
import jax.numpy as jnp
from jax import lax
from jax.experimental import pallas as pl
from jax.experimental.pallas import tpu as pltpu
from jax.experimental.pallas import tpu_sc as plsc


def _build(B, C, H, W, Hr, Wr):
    ph = (Hr - H) // 2
    pw = (Wr - W) // 2
    C2 = 2 * C

    NC, NS = 2, 16
    NW = NC * NS
    PZ = (B * C) // NW
    ZB = 2

    mesh = plsc.VectorSubcoreMesh(core_axis_name="c", subcore_axis_name="s")

    @functools.partial(
        pl.kernel,
        mesh=mesh,
        out_type=jax.ShapeDtypeStruct((B, C2, Hr, Wr), jnp.float32),
        scratch_types=[
            pltpu.VMEM((ZB, Hr, Wr), jnp.float32),
            pltpu.SemaphoreType.DMA,
        ],
        compiler_params=pltpu.CompilerParams(use_tc_tiling_on_sc=True),
    )
    def fill_zero_planes(zp_hbm, out_hbm, zbuf, sem):
        wid = lax.axis_index("s") * NC + lax.axis_index("c")
        pltpu.sync_copy(zp_hbm, zbuf)
        handles = []
        for i in range(PZ // ZB):
            p = wid * PZ + i * ZB
            b = p // C
            c = C + p % C
            handles.append(pltpu.async_copy(
                zbuf, out_hbm.at[b, pl.ds(c, ZB)], sem))
        for h in handles:
            h.wait()

    CB = 16

    def pad_body(x_ref, buf_ref, out_ref):
        del buf_ref
        for ch in range(CB):
            out_ref[0, ch] = jnp.zeros((Hr, Wr), jnp.float32)
            out_ref[0, ch, ph:ph + H, pw:pw + W] = x_ref[0, ch]

    pad_x = pl.pallas_call(
        pad_body,
        grid=(B, C // CB),
        in_specs=[
            pl.BlockSpec((1, CB, H, W), lambda b, g: (b, g, 0, 0)),
            pl.BlockSpec(memory_space=pl.ANY),
        ],
        out_specs=pl.BlockSpec((1, CB, Hr, Wr), lambda b, g: (b, g, 0, 0)),
        out_shape=jax.ShapeDtypeStruct((B, C2, Hr, Wr), jnp.float32),
        input_output_aliases={1: 0},
    )

    def run(x):
        zp = jnp.zeros((ZB, Hr, Wr), jnp.float32)
        buf = fill_zero_planes(zp)
        return pad_x(x, buf)

    return run


def kernel(x, reg):
    B, C, H, W = x.shape
    Hr, Wr = reg.shape[2], reg.shape[3]
    return _build(B, C, H, W, Hr, Wr)(x)


# speedup vs baseline: 5.1118x; 1.0003x over previous
import functools

import jax
import jax.numpy as jnp
from jax import lax
from jax.experimental import pallas as pl
from jax.experimental.pallas import tpu as pltpu
from jax.experimental.pallas import tpu_sc as plsc


def _build(B, C, H, W, Hr, Wr):
    ph = (Hr - H) // 2
    pw = (Wr - W) // 2
    C2 = 2 * C

    NC, NS = 2, 16
    NW = NC * NS
    PZ = (B * C) // NW
    ZB = 2

    mesh = plsc.VectorSubcoreMesh(core_axis_name="c", subcore_axis_name="s")

    @functools.partial(
        pl.kernel,
        mesh=mesh,
        out_type=jax.ShapeDtypeStruct((B, C2, Hr, Wr), jnp.float32),
        scratch_types=[
            pltpu.VMEM((ZB, Hr, Wr), jnp.float32),
            pltpu.SemaphoreType.DMA,
        ],
        compiler_params=pltpu.CompilerParams(use_tc_tiling_on_sc=True),
    )
    def fill_zero_planes(zp_hbm, out_hbm, zbuf, sem):
        wid = lax.axis_index("s") * NC + lax.axis_index("c")
        pltpu.sync_copy(zp_hbm, zbuf)
        handles = []
        for i in range(PZ // ZB):
            p = wid * PZ + i * ZB
            b = p // C
            c = C + p % C
            handles.append(pltpu.async_copy(
                zbuf, out_hbm.at[b, pl.ds(c, ZB)], sem))
        for h in handles:
            h.wait()

    CB = 16

    def pad_body(x_ref, buf_ref, out_ref):
        del buf_ref
        out_ref[0, :, :ph, :] = jnp.zeros((CB, ph, Wr), jnp.float32)
        out_ref[0, :, ph + H:, :] = jnp.zeros((CB, Hr - ph - H, Wr), jnp.float32)
        out_ref[0, :, ph:ph + H, :pw] = jnp.zeros((CB, H, pw), jnp.float32)
        out_ref[0, :, ph:ph + H, pw + W:] = jnp.zeros((CB, H, Wr - pw - W), jnp.float32)
        out_ref[0, :, ph:ph + H, pw:pw + W] = x_ref[0]

    pad_x = pl.pallas_call(
        pad_body,
        grid=(B, C // CB),
        in_specs=[
            pl.BlockSpec((1, CB, H, W), lambda b, g: (b, g, 0, 0)),
            pl.BlockSpec(memory_space=pl.ANY),
        ],
        out_specs=pl.BlockSpec((1, CB, Hr, Wr), lambda b, g: (b, g, 0, 0)),
        out_shape=jax.ShapeDtypeStruct((B, C2, Hr, Wr), jnp.float32),
        input_output_aliases={1: 0},
    )

    def run(x):
        zp = jnp.zeros((ZB, Hr, Wr), jnp.float32)
        buf = fill_zero_planes(zp)
        return pad_x(x, buf)

    return run


def kernel(x, reg):
    B, C, H, W = x.shape
    Hr, Wr = reg.shape[2], reg.shape[3]
    return _build(B, C, H, W, Hr, Wr)(x)


# speedup vs baseline: 5.2160x; 1.0204x over previous
import functools

import jax
import jax.numpy as jnp
from jax import lax
from jax.experimental import pallas as pl
from jax.experimental.pallas import tpu as pltpu
from jax.experimental.pallas import tpu_sc as plsc


def _build(B, C, H, W, Hr, Wr):
    ph = (Hr - H) // 2
    pw = (Wr - W) // 2
    C2 = 2 * C

    NC, NS = 2, 16
    NW = NC * NS
    PZ = (B * C) // NW
    ZB = 2

    mesh = plsc.VectorSubcoreMesh(core_axis_name="c", subcore_axis_name="s")

    @functools.partial(
        pl.kernel,
        mesh=mesh,
        out_type=jax.ShapeDtypeStruct((B, C2, Hr, Wr), jnp.float32),
        scratch_types=[
            pltpu.VMEM((ZB, Hr, Wr), jnp.float32),
            pltpu.SemaphoreType.DMA,
        ],
        compiler_params=pltpu.CompilerParams(use_tc_tiling_on_sc=True),
    )
    def fill_zero_planes(zp_hbm, out_hbm, zbuf, sem):
        wid = lax.axis_index("s") * NC + lax.axis_index("c")
        pltpu.sync_copy(zp_hbm, zbuf)
        handles = []
        for i in range(PZ // ZB):
            p = wid * PZ + i * ZB
            b = p // C
            c = C + p % C
            handles.append(pltpu.async_copy(
                zbuf, out_hbm.at[b, pl.ds(c, ZB)], sem))
        for h in handles:
            h.wait()

    CB = 32

    def pad_body(x_ref, buf_ref, out_ref):
        del buf_ref
        out_ref[0, :, :ph, :] = jnp.zeros((CB, ph, Wr), jnp.float32)
        out_ref[0, :, ph + H:, :] = jnp.zeros((CB, Hr - ph - H, Wr), jnp.float32)
        out_ref[0, :, ph:ph + H, :pw] = jnp.zeros((CB, H, pw), jnp.float32)
        out_ref[0, :, ph:ph + H, pw + W:] = jnp.zeros((CB, H, Wr - pw - W), jnp.float32)
        out_ref[0, :, ph:ph + H, pw:pw + W] = x_ref[0]

    pad_x = pl.pallas_call(
        pad_body,
        grid=(B, C // CB),
        in_specs=[
            pl.BlockSpec((1, CB, H, W), lambda b, g: (b, g, 0, 0)),
            pl.BlockSpec(memory_space=pl.ANY),
        ],
        out_specs=pl.BlockSpec((1, CB, Hr, Wr), lambda b, g: (b, g, 0, 0)),
        out_shape=jax.ShapeDtypeStruct((B, C2, Hr, Wr), jnp.float32),
        input_output_aliases={1: 0},
    )

    def run(x):
        zp = jnp.zeros((ZB, Hr, Wr), jnp.float32)
        buf = fill_zero_planes(zp)
        return pad_x(x, buf)

    return run


def kernel(x, reg):
    B, C, H, W = x.shape
    Hr, Wr = reg.shape[2], reg.shape[3]
    return _build(B, C, H, W, Hr, Wr)(x)
